# baseline (speedup 1.0000x reference)
import jax
import jax.numpy as jnp
from jax import lax
from jax.experimental import pallas as pl
from jax.experimental.pallas import tpu as pltpu

_ROWS = 32
_COLS = 1_000_000
_S = 800
_L = 1250
_CZ = 8
_UZ = 2
_NZ = _S // (_CZ * _UZ)
_CE = 40
_NE = _S // _CE

_ROT_A = (13, 15, 26, 6)
_ROT_B = (17, 29, 16, 24)
_KS = (0, 1, 0x1BD11BDA ^ 0 ^ 1)


def _threefry_bits_multi(xs):
    x0s = list(xs)
    x1s = [((x << 13) | (x >> 19)) ^ x for x in xs]
    first = True
    for i in range(5):
        rots = _ROT_A if i % 2 == 0 else _ROT_B
        for r in (rots[1:] if first else rots):
            x0s = [a + b for a, b in zip(x0s, x1s)]
            x1s = [(b << r) | (b >> (32 - r)) for b in x1s]
            x1s = [b ^ a for a, b in zip(x0s, x1s)]
        first = False
        k0 = jnp.uint32(_KS[(i + 1) % 3])
        k1 = jnp.uint32(_KS[(i + 2) % 3] + i + 1)
        x0s = [a + k0 for a in x0s]
        x1s = [b + k1 for b in x1s]
    return [a ^ b for a, b in zip(x0s, x1s)]


def _gumbel_softmax_kernel(inv_t_ref, x_ref, o_ref):
    row = pl.program_id(0)
    inv_t = inv_t_ref[0, 0]
    eps = jnp.float32(1e-20)
    one = jnp.float32(1.0)
    sub = lax.broadcasted_iota(jnp.uint32, (_CZ, _L), 0)
    lane = lax.broadcasted_iota(jnp.uint32, (_CZ, _L), 1)
    cvec = sub * jnp.uint32(_L) + lane
    base = jnp.uint32(row * _COLS + 1)

    def z_body(k, m_vec):
        s0 = k * (_CZ * _UZ)
        offs = [(s0 + j * _CZ).astype(jnp.uint32) * jnp.uint32(_L) + base
                for j in range(_UZ)]
        bits = _threefry_bits_multi([cvec + off for off in offs])
        us = [lax.bitcast_convert_type((b >> 9) | jnp.uint32(0x3F800000),
                                       jnp.float32) - one
              for b in bits]
        l1s = [jnp.log(u + eps) for u in us]
        gs = [-jnp.log(eps - l1) for l1 in l1s]
        zs = [(x_ref[0, pl.ds(s0 + j * _CZ, _CZ), :] + gs[j]) * inv_t
              for j in range(_UZ)]
        for j in range(_UZ):
            o_ref[0, pl.ds(s0 + j * _CZ, _CZ), :] = zs[j]
        for j in range(_UZ):
            m_vec = jnp.maximum(m_vec, zs[j])
        return m_vec

    m_vec = lax.fori_loop(
        0, _NZ, z_body, jnp.full((_CZ, _L), -jnp.inf, jnp.float32))
    m = jnp.max(m_vec)

    def e_body(k, s_vec):
        e = jnp.exp(o_ref[0, pl.ds(k * _CE, _CE), :] - m)
        o_ref[0, pl.ds(k * _CE, _CE), :] = e
        return s_vec + e

    s_vec = lax.fori_loop(
        0, _NE, e_body, jnp.zeros((_CE, _L), jnp.float32))
    inv_s = one / jnp.sum(s_vec)

    def scale_body(k, carry):
        o_ref[0, pl.ds(k * _CE, _CE), :] *= inv_s
        return carry

    lax.fori_loop(0, _NE, scale_body, jnp.float32(0.0))


def kernel(logits, temperature, use_gpu):
    del use_gpu
    inv_t = (jnp.float32(1.0)
             / jnp.asarray(temperature, jnp.float32)).reshape(1, 1)
    out = pl.pallas_call(
        _gumbel_softmax_kernel,
        grid=(_ROWS,),
        in_specs=[
            pl.BlockSpec(memory_space=pltpu.SMEM),
            pl.BlockSpec((1, _S, _L), lambda i: (i, 0, 0)),
        ],
        out_specs=pl.BlockSpec((1, _S, _L), lambda i: (i, 0, 0)),
        out_shape=jax.ShapeDtypeStruct((_ROWS, _S, _L), jnp.float32),
        compiler_params=pltpu.CompilerParams(
            dimension_semantics=("parallel",),
        ),
    )(inv_t, logits.reshape(_ROWS, _S, _L))
    return out.reshape(_ROWS, _COLS)


# speedup vs baseline: 1.0715x; 1.0715x over previous
import jax
import jax.numpy as jnp
from jax import lax
from jax.experimental import pallas as pl
from jax.experimental.pallas import tpu as pltpu

_ROWS = 32
_COLS = 1_000_000
_S = 1000
_L = 1000
_CZ = 8
_UZ = 5
_NZ = _S // (_CZ * _UZ)
_CE = 40
_NE = _S // _CE

_ROT_A = (13, 15, 26, 6)
_ROT_B = (17, 29, 16, 24)
_KS = (0, 1, 0x1BD11BDA ^ 0 ^ 1)


def _threefry_bits_multi(xs):
    x0s = list(xs)
    x1s = [((x << 13) | (x >> 19)) ^ x for x in xs]
    first = True
    for i in range(5):
        rots = _ROT_A if i % 2 == 0 else _ROT_B
        for r in (rots[1:] if first else rots):
            x0s = [a + b for a, b in zip(x0s, x1s)]
            x1s = [(b << r) | (b >> (32 - r)) for b in x1s]
            x1s = [b ^ a for a, b in zip(x0s, x1s)]
        first = False
        k0 = jnp.uint32(_KS[(i + 1) % 3])
        k1 = jnp.uint32(_KS[(i + 2) % 3] + i + 1)
        x0s = [a + k0 for a in x0s]
        x1s = [b + k1 for b in x1s]
    return [a ^ b for a, b in zip(x0s, x1s)]


def _gumbel_softmax_kernel(inv_t_ref, x_ref, o_ref):
    row = pl.program_id(0)
    inv_t = inv_t_ref[0, 0]
    eps = jnp.float32(1e-20)
    one = jnp.float32(1.0)
    sub = lax.broadcasted_iota(jnp.uint32, (_CZ, _L), 0)
    lane = lax.broadcasted_iota(jnp.uint32, (_CZ, _L), 1)
    cvec = sub * jnp.uint32(_L) + lane
    base = jnp.uint32(row * _COLS + 1)

    def z_body(k, m_vec):
        s0 = k * (_CZ * _UZ)
        offs = [(s0 + j * _CZ).astype(jnp.uint32) * jnp.uint32(_L) + base
                for j in range(_UZ)]
        bits = _threefry_bits_multi([cvec + off for off in offs])
        us = [lax.bitcast_convert_type((b >> 9) | jnp.uint32(0x3F800000),
                                       jnp.float32) - one
              for b in bits]
        l1s = [jnp.log(u + eps) for u in us]
        gs = [-jnp.log(eps - l1) for l1 in l1s]
        zs = [(x_ref[0, pl.ds(s0 + j * _CZ, _CZ), :] + gs[j]) * inv_t
              for j in range(_UZ)]
        for j in range(_UZ):
            o_ref[0, pl.ds(s0 + j * _CZ, _CZ), :] = zs[j]
        for j in range(_UZ):
            m_vec = jnp.maximum(m_vec, zs[j])
        return m_vec

    m_vec = lax.fori_loop(
        0, _NZ, z_body, jnp.full((_CZ, _L), -jnp.inf, jnp.float32))
    m = jnp.max(m_vec)

    def e_body(k, s_vec):
        e = jnp.exp(o_ref[0, pl.ds(k * _CE, _CE), :] - m)
        o_ref[0, pl.ds(k * _CE, _CE), :] = e
        return s_vec + e

    s_vec = lax.fori_loop(
        0, _NE, e_body, jnp.zeros((_CE, _L), jnp.float32))
    inv_s = one / jnp.sum(s_vec)

    def scale_body(k, carry):
        o_ref[0, pl.ds(k * _CE, _CE), :] *= inv_s
        return carry

    lax.fori_loop(0, _NE, scale_body, jnp.float32(0.0))


def kernel(logits, temperature, use_gpu):
    del use_gpu
    inv_t = (jnp.float32(1.0)
             / jnp.asarray(temperature, jnp.float32)).reshape(1, 1)
    out = pl.pallas_call(
        _gumbel_softmax_kernel,
        grid=(_ROWS,),
        in_specs=[
            pl.BlockSpec(memory_space=pltpu.SMEM),
            pl.BlockSpec((1, _S, _L), lambda i: (i, 0, 0)),
        ],
        out_specs=pl.BlockSpec((1, _S, _L), lambda i: (i, 0, 0)),
        out_shape=jax.ShapeDtypeStruct((_ROWS, _S, _L), jnp.float32),
        compiler_params=pltpu.CompilerParams(
            dimension_semantics=("parallel",),
        ),
    )(inv_t, logits.reshape(_ROWS, _S, _L))
    return out.reshape(_ROWS, _COLS)


# speedup vs baseline: 1.8325x; 1.7102x over previous
import jax
import jax.numpy as jnp
from jax import lax
from jax.experimental import pallas as pl
from jax.experimental.pallas import tpu as pltpu

_ROWS = 32
_COLS = 1_000_000
_RG = 8
_W = 4096
_NFULL = _COLS // _W
_TAIL = _COLS - _NFULL * _W
_NT = _NFULL + 1
_GRP = 32
_NGRP = 8

_ROT_A = (13, 15, 26, 6)
_ROT_B = (17, 29, 16, 24)
_KS = (0, 1, 0x1BD11BDA ^ 0 ^ 1)


def _threefry_bits(x1):
    x0 = x1
    x1 = ((x1 << 13) | (x1 >> 19)) ^ x0
    first = True
    for i in range(5):
        rots = _ROT_A if i % 2 == 0 else _ROT_B
        for r in (rots[1:] if first else rots):
            x0 = x0 + x1
            x1 = (x1 << r) | (x1 >> (32 - r))
            x1 = x1 ^ x0
        first = False
        x0 = x0 + jnp.uint32(_KS[(i + 1) % 3])
        x1 = x1 + jnp.uint32(_KS[(i + 2) % 3] + i + 1)
    return x0 ^ x1


def _gumbel_z(cvec, base_u32, x, inv_t, eps):
    bits = _threefry_bits(cvec + base_u32)
    fbits = (bits >> 9) | jnp.uint32(0x3F800000)
    u = lax.bitcast_convert_type(fbits, jnp.float32) - jnp.float32(1.0)
    g = -jnp.log(eps - jnp.log(u + eps))
    return (x + g) * inv_t


def _gumbel_softmax_kernel(inv_t_ref, x_hbm, o_hbm, zbuf, tailbuf, insems,
                           outsem):
    rg = pl.program_id(0)
    inv_t = inv_t_ref[0, 0]
    eps = jnp.float32(1e-20)
    r0 = rg * _RG

    def tile_w(t):
        return _W if t < _NFULL else _TAIL

    def in_copy(t):
        if t == _NFULL:
            return pltpu.make_async_copy(
                x_hbm.at[pl.ds(r0, _RG), pl.ds(t * _W, _TAIL)],
                tailbuf,
                insems.at[t // _GRP])
        return pltpu.make_async_copy(
            x_hbm.at[pl.ds(r0, _RG), pl.ds(t * _W, _W)],
            zbuf.at[t],
            insems.at[t // _GRP])

    def start_group(g):
        for t in range(g * _GRP, min((g + 1) * _GRP, _NT)):
            in_copy(t).start()

    def wait_group(g):
        for t in range(g * _GRP, min((g + 1) * _GRP, _NT)):
            in_copy(t).wait()

    sub = lax.broadcasted_iota(jnp.uint32, (_RG, _W), 0)
    lane = lax.broadcasted_iota(jnp.uint32, (_RG, _W), 1)
    cvec = sub * jnp.uint32(_COLS) + lane
    base0 = jnp.uint32(r0 * _COLS + 1)

    start_group(0)
    m_vec = jnp.full((_RG, 1024), -jnp.inf, jnp.float32)
    m_tail = jnp.full((_RG, 1), -jnp.inf, jnp.float32)
    for g in range(_NGRP):
        if g + 1 < _NGRP:
            start_group(g + 1)
        wait_group(g)
        n_in_g = min((g + 1) * _GRP, _NFULL) - g * _GRP

        def z_body(j, mv, g=g):
            t = g * _GRP + j
            base = base0 + (t * _W).astype(jnp.uint32)
            z = _gumbel_z(cvec, base, zbuf[t], inv_t, eps)
            zbuf[t] = z
            return jnp.maximum(
                jnp.maximum(mv, jnp.maximum(z[:, :1024], z[:, 1024:2048])),
                jnp.maximum(z[:, 2048:3072], z[:, 3072:]))

        m_vec = lax.fori_loop(0, n_in_g, z_body, m_vec)
        if g == _NGRP - 1:
            base = base0 + jnp.uint32(_NFULL * _W)
            zt = _gumbel_z(cvec[:, :_TAIL], base, tailbuf[...], inv_t, eps)
            tailbuf[...] = zt
            m_tail = jnp.max(zt, axis=1, keepdims=True)

    m = jnp.maximum(jnp.max(m_vec, axis=1, keepdims=True), m_tail)

    def e_body(j, sv):
        e = jnp.exp(zbuf[j] - m)
        zbuf[j] = e
        return sv + (e[:, :1024] + e[:, 1024:2048]) + (e[:, 2048:3072]
                                                       + e[:, 3072:])

    s_vec = lax.fori_loop(0, _NFULL, e_body,
                          jnp.zeros((_RG, 1024), jnp.float32))
    et = jnp.exp(tailbuf[...] - m)
    tailbuf[...] = et
    s = jnp.sum(s_vec, axis=1, keepdims=True) + jnp.sum(
        et, axis=1, keepdims=True)
    inv_s = jnp.float32(1.0) / s

    def out_copy_full(t):
        return pltpu.make_async_copy(
            zbuf.at[t, :, :],
            o_hbm.at[pl.ds(r0, _RG), pl.ds(t * _W, _W)],
            outsem)

    def out_copy_tail():
        return pltpu.make_async_copy(
            tailbuf,
            o_hbm.at[pl.ds(r0, _RG), pl.ds(_NFULL * _W, _TAIL)],
            outsem)

    def scale_body(j, carry):
        zbuf[j] = zbuf[j] * inv_s
        out_copy_full(j).start()
        return carry

    lax.fori_loop(0, _NFULL, scale_body, jnp.float32(0.0))
    tailbuf[...] = et * inv_s
    out_copy_tail().start()

    def out_wait(j, carry):
        out_copy_full(0).wait()
        return carry

    lax.fori_loop(0, _NFULL, out_wait, jnp.float32(0.0))
    out_copy_tail().wait()


def kernel(logits, temperature, use_gpu):
    del use_gpu
    inv_t = (jnp.float32(1.0)
             / jnp.asarray(temperature, jnp.float32)).reshape(1, 1)
    return pl.pallas_call(
        _gumbel_softmax_kernel,
        grid=(_ROWS // _RG,),
        in_specs=[
            pl.BlockSpec(memory_space=pltpu.SMEM),
            pl.BlockSpec(memory_space=pl.ANY),
        ],
        out_specs=pl.BlockSpec(memory_space=pl.ANY),
        out_shape=jax.ShapeDtypeStruct((_ROWS, _COLS), jnp.float32),
        scratch_shapes=[
            pltpu.VMEM((_NFULL, _RG, _W), jnp.float32),
            pltpu.VMEM((_RG, _TAIL), jnp.float32),
            pltpu.SemaphoreType.DMA((_NGRP,)),
            pltpu.SemaphoreType.DMA,
        ],
        compiler_params=pltpu.CompilerParams(
            dimension_semantics=("arbitrary",),
        ),
    )(inv_t, logits)
